# baseline (speedup 1.0000x reference)
import functools

import numpy as np
import jax
import jax.numpy as jnp
from jax import lax
from jax.experimental import pallas as pl
from jax.experimental.pallas import tpu as pltpu
from jax.experimental.pallas import tpu_sc as plsc

B, K, H = 64, 1024, 256
ESS_THRESHOLD = 0.5 * K
CH = 256
NCHUNK = K // CH
TINY = np.float32(np.finfo(np.float32).tiny)

_K1 = 2247515013
_K2 = 2545468385
_KS2 = (_K1 ^ _K2 ^ 0x1BD11BDA) & 0xFFFFFFFF


def _u(v):
    return jnp.uint32(v & 0xFFFFFFFF)


def _rotl(x, r):
    return (x << _u(r)) | (x >> _u(32 - r))


def _threefry_bits(cnt):
    x0 = jnp.full(cnt.shape, _u(_K1), jnp.uint32)
    x1 = cnt + _u(_K2)

    def rounds(x0, x1, rs):
        for r in rs:
            x0 = x0 + x1
            x1 = _rotl(x1, r)
            x1 = x0 ^ x1
        return x0, x1

    R0 = (13, 15, 26, 6)
    R1 = (17, 29, 16, 24)
    x0, x1 = rounds(x0, x1, R0)
    x0 = x0 + _u(_K2); x1 = x1 + _u(_KS2 + 1)
    x0, x1 = rounds(x0, x1, R1)
    x0 = x0 + _u(_KS2); x1 = x1 + _u(_K1 + 2)
    x0, x1 = rounds(x0, x1, R0)
    x0 = x0 + _u(_K1); x1 = x1 + _u(_K2 + 3)
    x0, x1 = rounds(x0, x1, R1)
    x0 = x0 + _u(_K2); x1 = x1 + _u(_KS2 + 4)
    x0, x1 = rounds(x0, x1, R0)
    x0 = x0 + _u(_KS2); x1 = x1 + _u(_K1 + 5)
    return x0 ^ x1


def _tc_body(lw_full_ref, lw_row_ref, lw_col_ref,
             anc_ref, nlw_ref, flag_ref, nlw_s, anc_s):
    b = pl.program_id(0)

    @pl.when(b == 0)
    def _():
        lw = lw_full_ref[...]
        m1 = jnp.max(lw, axis=1, keepdims=True)
        s1 = jnp.sum(jnp.exp(lw - m1), axis=1, keepdims=True)
        ln = lw - (jnp.log(s1) + m1)
        x2 = 2.0 * ln
        m2 = jnp.max(x2, axis=1, keepdims=True)
        s2 = jnp.sum(jnp.exp(x2 - m2), axis=1, keepdims=True)
        ess = jnp.exp(-(jnp.log(s2) + m2))
        n_resample = jnp.sum((ess < ESS_THRESHOLD).astype(jnp.int32))
        flag_ref[0] = (n_resample > 0).astype(jnp.int32)

    lw_row = lw_row_ref[0]
    prop = 0.5 * jnp.exp(lw_row) + np.float32(0.5 * (1.0 / K))
    logits = jnp.log(prop + np.float32(1e-10))
    d = lw_row - logits

    b_u = lax.convert_element_type(b * 1024, jnp.uint32)

    def chunk(ck, carry):
        base = lax.convert_element_type(ck, jnp.uint32) * _u(CH * 65536) + b_u
        kk = lax.broadcasted_iota(jnp.uint32, (CH, K), 0)
        jj = lax.broadcasted_iota(jnp.uint32, (CH, K), 1)
        cnt = kk * _u(65536) + jj + base
        bits = _threefry_bits(cnt)
        fb = (bits >> _u(9)) | _u(0x3F800000)
        f = lax.bitcast_convert_type(fb, jnp.float32) - 1.0
        u = jnp.maximum(f, TINY)
        g = -jnp.log(-jnp.log(u))
        z = g + logits
        m = jnp.max(z, axis=1, keepdims=True)
        jio = lax.broadcasted_iota(jnp.int32, (CH, K), 1)
        a = jnp.min(jnp.where(z == m, jio, K), axis=1, keepdims=True)
        dg = jnp.sum(jnp.where(jio == a, d, 0.0), axis=1, keepdims=True)
        off = ck * CH
        nlw_s[pl.ds(off, CH), :] = dg
        anc_s[pl.ds(off, CH), :] = a
        return carry

    lax.fori_loop(0, NCHUNK, chunk, 0)

    anyv = flag_ref[0] != 0
    nlw = nlw_s[...]
    m3 = jnp.max(nlw)
    lse = jnp.log(jnp.sum(jnp.exp(nlw - m3))) + m3
    lw_col = lw_col_ref[0]
    nlw_ref[...] = jnp.where(anyv, nlw - lse, lw_col).reshape(1, K, 1)
    kio = lax.broadcasted_iota(jnp.int32, (K, 1), 0)
    anc_ref[...] = (jnp.where(anyv, anc_s[...], kio) + b * 1024).reshape(1, K, 1)


_tc_call = pl.pallas_call(
    _tc_body,
    grid=(B,),
    in_specs=[
        pl.BlockSpec((B, K), lambda b: (0, 0)),
        pl.BlockSpec((1, 1, K), lambda b: (b, 0, 0)),
        pl.BlockSpec((1, K, 1), lambda b: (b, 0, 0)),
    ],
    out_specs=[
        pl.BlockSpec((1, K, 1), lambda b: (b, 0, 0)),
        pl.BlockSpec((1, K, 1), lambda b: (b, 0, 0)),
    ],
    out_shape=[
        jax.ShapeDtypeStruct((B, K, 1), jnp.int32),
        jax.ShapeDtypeStruct((B, K, 1), jnp.float32),
    ],
    scratch_shapes=[
        pltpu.SMEM((1,), jnp.int32),
        pltpu.VMEM((K, 1), jnp.float32),
        pltpu.VMEM((K, 1), jnp.int32),
    ],
    compiler_params=pltpu.CompilerParams(
        dimension_semantics=("arbitrary",)),
)


_NC, _NS = 2, 16
_NW = _NC * _NS
_RPW = (B * K) // _NW
_GCH = 128
_NG = _RPW // _GCH


def _sc_gather_body(table, idx, out, idx_v, buf0, buf1, sem0, sem1):
    wid = lax.axis_index("s") * _NC + lax.axis_index("c")
    base = wid * _RPW
    pltpu.sync_copy(idx.at[pl.ds(base, _RPW)], idx_v)
    bufs = (buf0, buf1)
    sems = (sem0, sem1)
    prev = None
    for i in range(_NG):
        cp = pltpu.async_copy(
            table.at[idx_v.at[pl.ds(i * _GCH, _GCH)]], bufs[i % 2], sems[i % 2])
        if prev is not None:
            pcp, pbuf, poff = prev
            pcp.wait()
            pltpu.sync_copy(pbuf, out.at[pl.ds(poff, _GCH)])
        prev = (cp, bufs[i % 2], base + i * _GCH)
    pcp, pbuf, poff = prev
    pcp.wait()
    pltpu.sync_copy(pbuf, out.at[pl.ds(poff, _GCH)])


@functools.lru_cache(maxsize=None)
def _make_sc_gather():
    return functools.partial(
        pl.kernel,
        out_type=jax.ShapeDtypeStruct((B * K, H), jnp.float32),
        mesh=plsc.VectorSubcoreMesh(core_axis_name="c", subcore_axis_name="s",
                                    num_cores=_NC, num_subcores=_NS),
        scratch_types=[
            pltpu.VMEM((_RPW,), jnp.int32),
            pltpu.VMEM((_GCH, H), jnp.float32),
            pltpu.VMEM((_GCH, H), jnp.float32),
            pltpu.SemaphoreType.DMA,
            pltpu.SemaphoreType.DMA,
        ],
    )(_sc_gather_body)


def kernel(particles, log_weights):
    anc3, nlw3 = _tc_call(log_weights,
                          log_weights.reshape(B, 1, K),
                          log_weights.reshape(B, K, 1))
    table = particles.reshape(B * K, H)
    out_flat = _make_sc_gather()(table, anc3.reshape(B * K))
    return out_flat.reshape(B, K, H), nlw3.reshape(B, K)


# speedup vs baseline: 1.0177x; 1.0177x over previous
import functools

import numpy as np
import jax
import jax.numpy as jnp
from jax import lax
from jax.experimental import pallas as pl
from jax.experimental.pallas import tpu as pltpu
from jax.experimental.pallas import tpu_sc as plsc

B, K, H = 64, 1024, 256
ESS_THRESHOLD = 0.5 * K
CH = 256
NCHUNK = K // CH
TJ = 128
TINY = np.float32(np.finfo(np.float32).tiny)

_K1 = 2247515013
_K2 = 2545468385
_KS2 = (_K1 ^ _K2 ^ 0x1BD11BDA) & 0xFFFFFFFF


def _u(v):
    return jnp.uint32(v & 0xFFFFFFFF)


def _rotl(x, r):
    return (x << _u(r)) | (x >> _u(32 - r))


def _threefry_bits(cnt):
    x0 = jnp.full(cnt.shape, _u(_K1), jnp.uint32)
    x1 = cnt + _u(_K2)

    def rounds(x0, x1, rs):
        for r in rs:
            x0 = x0 + x1
            x1 = _rotl(x1, r)
            x1 = x0 ^ x1
        return x0, x1

    R0 = (13, 15, 26, 6)
    R1 = (17, 29, 16, 24)
    x0, x1 = rounds(x0, x1, R0)
    x0 = x0 + _u(_K2); x1 = x1 + _u(_KS2 + 1)
    x0, x1 = rounds(x0, x1, R1)
    x0 = x0 + _u(_KS2); x1 = x1 + _u(_K1 + 2)
    x0, x1 = rounds(x0, x1, R0)
    x0 = x0 + _u(_K1); x1 = x1 + _u(_K2 + 3)
    x0, x1 = rounds(x0, x1, R1)
    x0 = x0 + _u(_K2); x1 = x1 + _u(_KS2 + 4)
    x0, x1 = rounds(x0, x1, R0)
    x0 = x0 + _u(_KS2); x1 = x1 + _u(_K1 + 5)
    return x0 ^ x1


def _tc_body(lw_full_ref, lw_row_ref, lw_col_ref,
             anc_ref, nlw_ref, flag_ref, nlw_s, anc_s):
    b = pl.program_id(0)

    @pl.when(b == 0)
    def _():
        lw = lw_full_ref[...]
        m1 = jnp.max(lw, axis=1, keepdims=True)
        s1 = jnp.sum(jnp.exp(lw - m1), axis=1, keepdims=True)
        ln = lw - (jnp.log(s1) + m1)
        x2 = 2.0 * ln
        m2 = jnp.max(x2, axis=1, keepdims=True)
        s2 = jnp.sum(jnp.exp(x2 - m2), axis=1, keepdims=True)
        ess = jnp.exp(-(jnp.log(s2) + m2))
        n_resample = jnp.sum((ess < ESS_THRESHOLD).astype(jnp.int32))
        flag_ref[0] = (n_resample > 0).astype(jnp.int32)

    lw_row = lw_row_ref[0]
    prop = 0.5 * jnp.exp(lw_row) + np.float32(0.5 * (1.0 / K))
    logits = jnp.log(prop + np.float32(1e-10))
    d = lw_row - logits

    b_u = lax.convert_element_type(b * 1024, jnp.uint32)

    def chunk(ck, carry):
        base = lax.convert_element_type(ck, jnp.uint32) * _u(CH * 65536) + b_u
        kk = lax.broadcasted_iota(jnp.uint32, (CH, TJ), 0) * _u(65536)
        lane = lax.broadcasted_iota(jnp.uint32, (CH, TJ), 1)
        cnt0 = kk + lane + base
        lane_i = lax.broadcasted_iota(jnp.int32, (CH, TJ), 1)
        m = None
        for t in range(K // TJ):
            bits = _threefry_bits(cnt0 + _u(t * TJ))
            fb = (bits >> _u(9)) | _u(0x3F800000)
            f = lax.bitcast_convert_type(fb, jnp.float32) - 1.0
            u = jnp.maximum(f, TINY)
            g = -jnp.log(-jnp.log(u))
            z = g + logits[:, t * TJ:(t + 1) * TJ]
            d_t = d[:, t * TJ:(t + 1) * TJ]
            if m is None:
                m, jidx, dgr = z, lane_i, jnp.broadcast_to(d_t, (CH, TJ))
            else:
                gt = z > m
                m = jnp.where(gt, z, m)
                jidx = jnp.where(gt, lane_i + t * TJ, jidx)
                dgr = jnp.where(gt, d_t, dgr)
        mfin = jnp.max(m, axis=1, keepdims=True)
        a = jnp.min(jnp.where(m == mfin, jidx, K), axis=1, keepdims=True)
        dg = jnp.sum(jnp.where(jidx == a, dgr, 0.0), axis=1, keepdims=True)
        off = ck * CH
        nlw_s[pl.ds(off, CH), :] = dg
        anc_s[pl.ds(off, CH), :] = a
        return carry

    lax.fori_loop(0, NCHUNK, chunk, 0)

    anyv = flag_ref[0] != 0
    nlw = nlw_s[...]
    m3 = jnp.max(nlw)
    lse = jnp.log(jnp.sum(jnp.exp(nlw - m3))) + m3
    lw_col = lw_col_ref[0]
    nlw_ref[...] = jnp.where(anyv, nlw - lse, lw_col).reshape(1, K, 1)
    kio = lax.broadcasted_iota(jnp.int32, (K, 1), 0)
    anc_ref[...] = (jnp.where(anyv, anc_s[...], kio) + b * 1024).reshape(1, K, 1)


_tc_call = pl.pallas_call(
    _tc_body,
    grid=(B,),
    in_specs=[
        pl.BlockSpec((B, K), lambda b: (0, 0)),
        pl.BlockSpec((1, 1, K), lambda b: (b, 0, 0)),
        pl.BlockSpec((1, K, 1), lambda b: (b, 0, 0)),
    ],
    out_specs=[
        pl.BlockSpec((1, K, 1), lambda b: (b, 0, 0)),
        pl.BlockSpec((1, K, 1), lambda b: (b, 0, 0)),
    ],
    out_shape=[
        jax.ShapeDtypeStruct((B, K, 1), jnp.int32),
        jax.ShapeDtypeStruct((B, K, 1), jnp.float32),
    ],
    scratch_shapes=[
        pltpu.SMEM((1,), jnp.int32),
        pltpu.VMEM((K, 1), jnp.float32),
        pltpu.VMEM((K, 1), jnp.int32),
    ],
    compiler_params=pltpu.CompilerParams(
        dimension_semantics=("arbitrary",)),
)


_NC, _NS = 2, 16
_NW = _NC * _NS
_RPW = (B * K) // _NW
_GCH = 128
_NG = _RPW // _GCH


def _sc_gather_body(table, idx, out, idx_v, buf0, buf1, sem0, sem1):
    wid = lax.axis_index("s") * _NC + lax.axis_index("c")
    base = wid * _RPW
    pltpu.sync_copy(idx.at[pl.ds(base, _RPW)], idx_v)
    bufs = (buf0, buf1)
    sems = (sem0, sem1)
    prev = None
    for i in range(_NG):
        cp = pltpu.async_copy(
            table.at[idx_v.at[pl.ds(i * _GCH, _GCH)]], bufs[i % 2], sems[i % 2])
        if prev is not None:
            pcp, pbuf, poff = prev
            pcp.wait()
            pltpu.sync_copy(pbuf, out.at[pl.ds(poff, _GCH)])
        prev = (cp, bufs[i % 2], base + i * _GCH)
    pcp, pbuf, poff = prev
    pcp.wait()
    pltpu.sync_copy(pbuf, out.at[pl.ds(poff, _GCH)])


@functools.lru_cache(maxsize=None)
def _make_sc_gather():
    return functools.partial(
        pl.kernel,
        out_type=jax.ShapeDtypeStruct((B * K, H), jnp.float32),
        mesh=plsc.VectorSubcoreMesh(core_axis_name="c", subcore_axis_name="s",
                                    num_cores=_NC, num_subcores=_NS),
        scratch_types=[
            pltpu.VMEM((_RPW,), jnp.int32),
            pltpu.VMEM((_GCH, H), jnp.float32),
            pltpu.VMEM((_GCH, H), jnp.float32),
            pltpu.SemaphoreType.DMA,
            pltpu.SemaphoreType.DMA,
        ],
    )(_sc_gather_body)


def kernel(particles, log_weights):
    anc3, nlw3 = _tc_call(log_weights,
                          log_weights.reshape(B, 1, K),
                          log_weights.reshape(B, K, 1))
    table = particles.reshape(B * K, H)
    out_flat = _make_sc_gather()(table, anc3.reshape(B * K))
    return out_flat.reshape(B, K, H), nlw3.reshape(B, K)


# speedup vs baseline: 1.0476x; 1.0293x over previous
import functools

import numpy as np
import jax
import jax.numpy as jnp
from jax import lax
from jax.experimental import pallas as pl
from jax.experimental.pallas import tpu as pltpu
from jax.experimental.pallas import tpu_sc as plsc

B, K, H = 64, 1024, 256
ESS_THRESHOLD = 0.5 * K
CH = 1024
NCHUNK = K // CH
TJ = 128
TINY = np.float32(np.finfo(np.float32).tiny)

_K1 = 2247515013
_K2 = 2545468385
_KS2 = (_K1 ^ _K2 ^ 0x1BD11BDA) & 0xFFFFFFFF


def _u(v):
    return jnp.uint32(v & 0xFFFFFFFF)


def _rotl(x, r):
    return (x << _u(r)) | (x >> _u(32 - r))


def _threefry_bits(cnt):
    x0 = jnp.full(cnt.shape, _u(_K1), jnp.uint32)
    x1 = cnt + _u(_K2)

    def rounds(x0, x1, rs):
        for r in rs:
            x0 = x0 + x1
            x1 = _rotl(x1, r)
            x1 = x0 ^ x1
        return x0, x1

    R0 = (13, 15, 26, 6)
    R1 = (17, 29, 16, 24)
    x0, x1 = rounds(x0, x1, R0)
    x0 = x0 + _u(_K2); x1 = x1 + _u(_KS2 + 1)
    x0, x1 = rounds(x0, x1, R1)
    x0 = x0 + _u(_KS2); x1 = x1 + _u(_K1 + 2)
    x0, x1 = rounds(x0, x1, R0)
    x0 = x0 + _u(_K1); x1 = x1 + _u(_K2 + 3)
    x0, x1 = rounds(x0, x1, R1)
    x0 = x0 + _u(_K2); x1 = x1 + _u(_KS2 + 4)
    x0, x1 = rounds(x0, x1, R0)
    x0 = x0 + _u(_KS2); x1 = x1 + _u(_K1 + 5)
    return x0 ^ x1


def _tc_body(lw_full_ref, lw_row_ref, lw_col_ref,
             anc_ref, nlw_ref, flag_ref, nlw_s, anc_s):
    b = pl.program_id(0)

    @pl.when(b == 0)
    def _():
        lw = lw_full_ref[...]
        m1 = jnp.max(lw, axis=1, keepdims=True)
        s1 = jnp.sum(jnp.exp(lw - m1), axis=1, keepdims=True)
        ln = lw - (jnp.log(s1) + m1)
        x2 = 2.0 * ln
        m2 = jnp.max(x2, axis=1, keepdims=True)
        s2 = jnp.sum(jnp.exp(x2 - m2), axis=1, keepdims=True)
        ess = jnp.exp(-(jnp.log(s2) + m2))
        n_resample = jnp.sum((ess < ESS_THRESHOLD).astype(jnp.int32))
        flag_ref[0] = (n_resample > 0).astype(jnp.int32)

    lw_row = lw_row_ref[0]
    prop = 0.5 * jnp.exp(lw_row) + np.float32(0.5 * (1.0 / K))
    logits = jnp.log(prop + np.float32(1e-10))
    d = lw_row - logits

    b_u = lax.convert_element_type(b * 1024, jnp.uint32)

    def chunk(ck, carry):
        base = lax.convert_element_type(ck, jnp.uint32) * _u(CH * 65536) + b_u
        kk = lax.broadcasted_iota(jnp.uint32, (CH, TJ), 0) * _u(65536)
        lane = lax.broadcasted_iota(jnp.uint32, (CH, TJ), 1)
        cnt0 = kk + lane + base
        lane_i = lax.broadcasted_iota(jnp.int32, (CH, TJ), 1)
        m = None
        for t in range(K // TJ):
            bits = _threefry_bits(cnt0 + _u(t * TJ))
            fb = (bits >> _u(9)) | _u(0x3F800000)
            f = lax.bitcast_convert_type(fb, jnp.float32) - 1.0
            u = jnp.maximum(f, TINY)
            g = -jnp.log(-jnp.log(u))
            z = g + logits[:, t * TJ:(t + 1) * TJ]
            d_t = d[:, t * TJ:(t + 1) * TJ]
            if m is None:
                m, jidx, dgr = z, lane_i, jnp.broadcast_to(d_t, (CH, TJ))
            else:
                gt = z > m
                m = jnp.where(gt, z, m)
                jidx = jnp.where(gt, lane_i + t * TJ, jidx)
                dgr = jnp.where(gt, d_t, dgr)
        mfin = jnp.max(m, axis=1, keepdims=True)
        a = jnp.min(jnp.where(m == mfin, jidx, K), axis=1, keepdims=True)
        dg = jnp.sum(jnp.where(jidx == a, dgr, 0.0), axis=1, keepdims=True)
        off = ck * CH
        nlw_s[pl.ds(off, CH), :] = dg
        anc_s[pl.ds(off, CH), :] = a
        return carry

    lax.fori_loop(0, NCHUNK, chunk, 0)

    anyv = flag_ref[0] != 0
    nlw = nlw_s[...]
    m3 = jnp.max(nlw)
    lse = jnp.log(jnp.sum(jnp.exp(nlw - m3))) + m3
    lw_col = lw_col_ref[0]
    nlw_ref[...] = jnp.where(anyv, nlw - lse, lw_col).reshape(1, K, 1)
    kio = lax.broadcasted_iota(jnp.int32, (K, 1), 0)
    anc_ref[...] = (jnp.where(anyv, anc_s[...], kio) + b * 1024).reshape(1, K, 1)


_tc_call = pl.pallas_call(
    _tc_body,
    grid=(B,),
    in_specs=[
        pl.BlockSpec((B, K), lambda b: (0, 0)),
        pl.BlockSpec((1, 1, K), lambda b: (b, 0, 0)),
        pl.BlockSpec((1, K, 1), lambda b: (b, 0, 0)),
    ],
    out_specs=[
        pl.BlockSpec((1, K, 1), lambda b: (b, 0, 0)),
        pl.BlockSpec((1, K, 1), lambda b: (b, 0, 0)),
    ],
    out_shape=[
        jax.ShapeDtypeStruct((B, K, 1), jnp.int32),
        jax.ShapeDtypeStruct((B, K, 1), jnp.float32),
    ],
    scratch_shapes=[
        pltpu.SMEM((1,), jnp.int32),
        pltpu.VMEM((K, 1), jnp.float32),
        pltpu.VMEM((K, 1), jnp.int32),
    ],
    compiler_params=pltpu.CompilerParams(
        dimension_semantics=("arbitrary",)),
)


_NC, _NS = 2, 16
_NW = _NC * _NS
_RPW = (B * K) // _NW
_GCH = 128
_NG = _RPW // _GCH


def _sc_gather_body(table, idx, out, idx_v, buf0, buf1, sem0, sem1):
    wid = lax.axis_index("s") * _NC + lax.axis_index("c")
    base = wid * _RPW
    pltpu.sync_copy(idx.at[pl.ds(base, _RPW)], idx_v)
    bufs = (buf0, buf1)
    sems = (sem0, sem1)
    prev = None
    for i in range(_NG):
        cp = pltpu.async_copy(
            table.at[idx_v.at[pl.ds(i * _GCH, _GCH)]], bufs[i % 2], sems[i % 2])
        if prev is not None:
            pcp, pbuf, poff = prev
            pcp.wait()
            pltpu.sync_copy(pbuf, out.at[pl.ds(poff, _GCH)])
        prev = (cp, bufs[i % 2], base + i * _GCH)
    pcp, pbuf, poff = prev
    pcp.wait()
    pltpu.sync_copy(pbuf, out.at[pl.ds(poff, _GCH)])


@functools.lru_cache(maxsize=None)
def _make_sc_gather():
    return functools.partial(
        pl.kernel,
        out_type=jax.ShapeDtypeStruct((B * K, H), jnp.float32),
        mesh=plsc.VectorSubcoreMesh(core_axis_name="c", subcore_axis_name="s",
                                    num_cores=_NC, num_subcores=_NS),
        scratch_types=[
            pltpu.VMEM((_RPW,), jnp.int32),
            pltpu.VMEM((_GCH, H), jnp.float32),
            pltpu.VMEM((_GCH, H), jnp.float32),
            pltpu.SemaphoreType.DMA,
            pltpu.SemaphoreType.DMA,
        ],
    )(_sc_gather_body)


def kernel(particles, log_weights):
    anc3, nlw3 = _tc_call(log_weights,
                          log_weights.reshape(B, 1, K),
                          log_weights.reshape(B, K, 1))
    table = particles.reshape(B * K, H)
    out_flat = _make_sc_gather()(table, anc3.reshape(B * K))
    return out_flat.reshape(B, K, H), nlw3.reshape(B, K)


# speedup vs baseline: 1.0547x; 1.0068x over previous
import functools

import numpy as np
import jax
import jax.numpy as jnp
from jax import lax
from jax.experimental import pallas as pl
from jax.experimental.pallas import tpu as pltpu
from jax.experimental.pallas import tpu_sc as plsc

B, K, H = 64, 1024, 256
ESS_THRESHOLD = 0.5 * K
CH = 1024
NCHUNK = K // CH
TJ = 128
NB = 16
TINY = np.float32(np.finfo(np.float32).tiny)

_K1 = 2247515013
_K2 = 2545468385
_KS2 = (_K1 ^ _K2 ^ 0x1BD11BDA) & 0xFFFFFFFF


def _u(v):
    return jnp.uint32(v & 0xFFFFFFFF)


def _rotl(x, r):
    return (x << _u(r)) | (x >> _u(32 - r))


def _threefry_bits(cnt):
    x0 = jnp.full(cnt.shape, _u(_K1), jnp.uint32)
    x1 = cnt + _u(_K2)

    def rounds(x0, x1, rs):
        for r in rs:
            x0 = x0 + x1
            x1 = _rotl(x1, r)
            x1 = x0 ^ x1
        return x0, x1

    R0 = (13, 15, 26, 6)
    R1 = (17, 29, 16, 24)
    x0, x1 = rounds(x0, x1, R0)
    x0 = x0 + _u(_K2); x1 = x1 + _u(_KS2 + 1)
    x0, x1 = rounds(x0, x1, R1)
    x0 = x0 + _u(_KS2); x1 = x1 + _u(_K1 + 2)
    x0, x1 = rounds(x0, x1, R0)
    x0 = x0 + _u(_K1); x1 = x1 + _u(_K2 + 3)
    x0, x1 = rounds(x0, x1, R1)
    x0 = x0 + _u(_K2); x1 = x1 + _u(_KS2 + 4)
    x0, x1 = rounds(x0, x1, R0)
    x0 = x0 + _u(_KS2); x1 = x1 + _u(_K1 + 5)
    return x0 ^ x1


def _tc_body(lw_full_ref, lw_col_ref, anc_ref, nlw_ref, flag_ref):
    pid = pl.program_id(0)

    @pl.when(pid == 0)
    def _():
        lw = lw_full_ref[...]
        m1 = jnp.max(lw, axis=1, keepdims=True)
        s1 = jnp.sum(jnp.exp(lw - m1), axis=1, keepdims=True)
        ln = lw - (jnp.log(s1) + m1)
        x2 = 2.0 * ln
        m2 = jnp.max(x2, axis=1, keepdims=True)
        s2 = jnp.sum(jnp.exp(x2 - m2), axis=1, keepdims=True)
        ess = jnp.exp(-(jnp.log(s2) + m2))
        n = jnp.sum((ess < ESS_THRESHOLD).astype(jnp.int32))
        flag_ref[0] = (n > 0).astype(jnp.int32)

    anyv = flag_ref[0] != 0
    kk = lax.broadcasted_iota(jnp.uint32, (CH, TJ), 0) * _u(65536)
    lane = lax.broadcasted_iota(jnp.uint32, (CH, TJ), 1)
    cnt00 = kk + lane
    lane_i = lax.broadcasted_iota(jnp.int32, (CH, TJ), 1)
    kio = lax.broadcasted_iota(jnp.int32, (K, 1), 0)

    def batch(bi, carry):
        bb = pid * NB + bi
        lw_row = lw_full_ref[pl.ds(bb, 1), :]
        prop = 0.5 * jnp.exp(lw_row) + np.float32(0.5 * (1.0 / K))
        logits = jnp.log(prop + np.float32(1e-10))
        d = lw_row - logits

        cnt0 = cnt00 + lax.convert_element_type(bb * 1024, jnp.uint32)
        m = None
        for t in range(K // TJ):
            bits = _threefry_bits(cnt0 + _u(t * TJ))
            fb = (bits >> _u(9)) | _u(0x3F800000)
            f = lax.bitcast_convert_type(fb, jnp.float32) - 1.0
            u = jnp.maximum(f, TINY)
            g = -jnp.log(-jnp.log(u))
            z = g + logits[:, t * TJ:(t + 1) * TJ]
            d_t = d[:, t * TJ:(t + 1) * TJ]
            if m is None:
                m, jidx, dgr = z, lane_i, jnp.broadcast_to(d_t, (CH, TJ))
            else:
                gt = z > m
                m = jnp.where(gt, z, m)
                jidx = jnp.where(gt, lane_i + t * TJ, jidx)
                dgr = jnp.where(gt, d_t, dgr)
        mfin = jnp.max(m, axis=1, keepdims=True)
        a = jnp.min(jnp.where(m == mfin, jidx, K), axis=1, keepdims=True)
        dg = jnp.sum(jnp.where(jidx == a, dgr, 0.0), axis=1, keepdims=True)

        m3 = jnp.max(dg)
        lse = jnp.log(jnp.sum(jnp.exp(dg - m3))) + m3
        lw_col = lw_col_ref[pl.ds(bi, 1)].reshape(K, 1)
        nlw_ref[pl.ds(bi, 1)] = jnp.where(anyv, dg - lse,
                                          lw_col).reshape(1, K, 1)
        anc_ref[pl.ds(bi, 1)] = (jnp.where(anyv, a, kio)
                                 + bb * 1024).reshape(1, K, 1)
        return carry

    lax.fori_loop(0, NB, batch, 0)


_tc_call = pl.pallas_call(
    _tc_body,
    grid=(B // NB,),
    in_specs=[
        pl.BlockSpec((B, K), lambda i: (0, 0)),
        pl.BlockSpec((NB, K, 1), lambda i: (i, 0, 0)),
    ],
    out_specs=[
        pl.BlockSpec((NB, K, 1), lambda i: (i, 0, 0)),
        pl.BlockSpec((NB, K, 1), lambda i: (i, 0, 0)),
    ],
    out_shape=[
        jax.ShapeDtypeStruct((B, K, 1), jnp.int32),
        jax.ShapeDtypeStruct((B, K, 1), jnp.float32),
    ],
    scratch_shapes=[
        pltpu.SMEM((1,), jnp.int32),
    ],
    compiler_params=pltpu.CompilerParams(
        dimension_semantics=("arbitrary",)),
)


_NC, _NS = 2, 16
_NW = _NC * _NS
_RPW = (B * K) // _NW
_GCH = 128
_NG = _RPW // _GCH


def _sc_gather_body(table, idx, out, idx_v, buf0, buf1, sem0, sem1):
    wid = lax.axis_index("s") * _NC + lax.axis_index("c")
    base = wid * _RPW
    pltpu.sync_copy(idx.at[pl.ds(base, _RPW)], idx_v)
    bufs = (buf0, buf1)
    sems = (sem0, sem1)
    prev = None
    for i in range(_NG):
        cp = pltpu.async_copy(
            table.at[idx_v.at[pl.ds(i * _GCH, _GCH)]], bufs[i % 2], sems[i % 2])
        if prev is not None:
            pcp, pbuf, poff = prev
            pcp.wait()
            pltpu.sync_copy(pbuf, out.at[pl.ds(poff, _GCH)])
        prev = (cp, bufs[i % 2], base + i * _GCH)
    pcp, pbuf, poff = prev
    pcp.wait()
    pltpu.sync_copy(pbuf, out.at[pl.ds(poff, _GCH)])


@functools.lru_cache(maxsize=None)
def _make_sc_gather():
    return functools.partial(
        pl.kernel,
        out_type=jax.ShapeDtypeStruct((B * K, H), jnp.float32),
        mesh=plsc.VectorSubcoreMesh(core_axis_name="c", subcore_axis_name="s",
                                    num_cores=_NC, num_subcores=_NS),
        scratch_types=[
            pltpu.VMEM((_RPW,), jnp.int32),
            pltpu.VMEM((_GCH, H), jnp.float32),
            pltpu.VMEM((_GCH, H), jnp.float32),
            pltpu.SemaphoreType.DMA,
            pltpu.SemaphoreType.DMA,
        ],
    )(_sc_gather_body)


def kernel(particles, log_weights):
    anc3, nlw3 = _tc_call(log_weights, log_weights.reshape(B, K, 1))
    table = particles.reshape(B * K, H)
    out_flat = _make_sc_gather()(table, anc3.reshape(B * K))
    return out_flat.reshape(B, K, H), nlw3.reshape(B, K)


# speedup vs baseline: 1.0659x; 1.0107x over previous
import functools

import numpy as np
import jax
import jax.numpy as jnp
from jax import lax
from jax.experimental import pallas as pl
from jax.experimental.pallas import tpu as pltpu
from jax.experimental.pallas import tpu_sc as plsc

B, K, H = 64, 1024, 256
ESS_THRESHOLD = 0.5 * K
CH = 1024
NCHUNK = K // CH
TJ = 128
NB = 16
TINY = np.float32(np.finfo(np.float32).tiny)

_K1 = 2247515013
_K2 = 2545468385
_KS2 = (_K1 ^ _K2 ^ 0x1BD11BDA) & 0xFFFFFFFF


def _u(v):
    return jnp.uint32(v & 0xFFFFFFFF)


def _rotl(x, r):
    return (x << _u(r)) | (x >> _u(32 - r))


def _threefry_bits(x1):
    x0 = jnp.full(x1.shape, _u(_K1), jnp.uint32)

    def rounds(x0, x1, rs):
        for r in rs:
            x0 = x0 + x1
            x1 = _rotl(x1, r)
            x1 = x0 ^ x1
        return x0, x1

    R0 = (13, 15, 26, 6)
    R1 = (17, 29, 16, 24)
    x0, x1 = rounds(x0, x1, R0)
    x0 = x0 + _u(_K2); x1 = x1 + _u(_KS2 + 1)
    x0, x1 = rounds(x0, x1, R1)
    x0 = x0 + _u(_KS2); x1 = x1 + _u(_K1 + 2)
    x0, x1 = rounds(x0, x1, R0)
    x0 = x0 + _u(_K1); x1 = x1 + _u(_K2 + 3)
    x0, x1 = rounds(x0, x1, R1)
    x0 = x0 + _u(_K2); x1 = x1 + _u(_KS2 + 4)
    x0, x1 = rounds(x0, x1, R0)
    x0 = x0 + _u(_KS2); x1 = x1 + _u(_K1 + 5)
    return x0 ^ x1


def _tc_body(lw_full_ref, lw_col_ref, anc_ref, nlw_ref, flag_ref):
    pid = pl.program_id(0)

    @pl.when(pid == 0)
    def _():
        lw = lw_full_ref[...]
        m1 = jnp.max(lw, axis=1, keepdims=True)
        s1 = jnp.sum(jnp.exp(lw - m1), axis=1, keepdims=True)
        ln = lw - (jnp.log(s1) + m1)
        x2 = 2.0 * ln
        m2 = jnp.max(x2, axis=1, keepdims=True)
        s2 = jnp.sum(jnp.exp(x2 - m2), axis=1, keepdims=True)
        ess = jnp.exp(-(jnp.log(s2) + m2))
        n = jnp.sum((ess < ESS_THRESHOLD).astype(jnp.int32))
        flag_ref[0] = (n > 0).astype(jnp.int32)

    anyv = flag_ref[0] != 0
    kk = lax.broadcasted_iota(jnp.uint32, (CH, TJ), 0) * _u(65536)
    lane = lax.broadcasted_iota(jnp.uint32, (CH, TJ), 1)
    cnt00 = kk + lane
    lane_i = lax.broadcasted_iota(jnp.int32, (CH, TJ), 1)
    kio = lax.broadcasted_iota(jnp.int32, (K, 1), 0)

    def batch(bi, carry):
        bb = pid * NB + bi
        lw_row = lw_full_ref[pl.ds(bb, 1), :]
        prop = 0.5 * jnp.exp(lw_row) + np.float32(0.5 * (1.0 / K))
        logits = jnp.log(prop + np.float32(1e-10))
        d = lw_row - logits

        boff = lax.convert_element_type(bb * 1024, jnp.uint32)
        m = None
        for t in range(K // TJ):
            bits = _threefry_bits(cnt00 + (boff + _u(t * TJ + _K2)))
            fb = (bits >> _u(9)) | _u(0x3F800000)
            f = lax.bitcast_convert_type(fb, jnp.float32) - 1.0
            u = jnp.maximum(f, TINY)
            y = jnp.log(-jnp.log(u)) - logits[:, t * TJ:(t + 1) * TJ]
            d_t = d[:, t * TJ:(t + 1) * TJ]
            if m is None:
                m, jidx, dgr = y, lane_i, jnp.broadcast_to(d_t, (CH, TJ))
            else:
                lt = y < m
                m = jnp.where(lt, y, m)
                jidx = jnp.where(lt, lane_i + t * TJ, jidx)
                dgr = jnp.where(lt, d_t, dgr)
        mfin = jnp.min(m, axis=1, keepdims=True)
        a = jnp.min(jnp.where(m == mfin, jidx, K), axis=1, keepdims=True)
        dg = jnp.sum(jnp.where(jidx == a, dgr, 0.0), axis=1, keepdims=True)

        m3 = jnp.max(dg)
        lse = jnp.log(jnp.sum(jnp.exp(dg - m3))) + m3
        lw_col = lw_col_ref[pl.ds(bi, 1)].reshape(K, 1)
        nlw_ref[pl.ds(bi, 1)] = jnp.where(anyv, dg - lse,
                                          lw_col).reshape(1, K, 1)
        anc_ref[pl.ds(bi, 1)] = (jnp.where(anyv, a, kio)
                                 + bb * 1024).reshape(1, K, 1)
        return carry

    lax.fori_loop(0, NB, batch, 0)


_tc_call = pl.pallas_call(
    _tc_body,
    grid=(B // NB,),
    in_specs=[
        pl.BlockSpec((B, K), lambda i: (0, 0)),
        pl.BlockSpec((NB, K, 1), lambda i: (i, 0, 0)),
    ],
    out_specs=[
        pl.BlockSpec((NB, K, 1), lambda i: (i, 0, 0)),
        pl.BlockSpec((NB, K, 1), lambda i: (i, 0, 0)),
    ],
    out_shape=[
        jax.ShapeDtypeStruct((B, K, 1), jnp.int32),
        jax.ShapeDtypeStruct((B, K, 1), jnp.float32),
    ],
    scratch_shapes=[
        pltpu.SMEM((1,), jnp.int32),
    ],
    compiler_params=pltpu.CompilerParams(
        dimension_semantics=("arbitrary",)),
)


_NC, _NS = 2, 16
_NW = _NC * _NS
_RPW = (B * K) // _NW
_GCH = 128
_NG = _RPW // _GCH


def _sc_gather_body(table, idx, out, idx_v, buf0, buf1, sem0, sem1):
    wid = lax.axis_index("s") * _NC + lax.axis_index("c")
    base = wid * _RPW
    pltpu.sync_copy(idx.at[pl.ds(base, _RPW)], idx_v)
    bufs = (buf0, buf1)
    sems = (sem0, sem1)
    prev = None
    for i in range(_NG):
        cp = pltpu.async_copy(
            table.at[idx_v.at[pl.ds(i * _GCH, _GCH)]], bufs[i % 2], sems[i % 2])
        if prev is not None:
            pcp, pbuf, poff = prev
            pcp.wait()
            pltpu.sync_copy(pbuf, out.at[pl.ds(poff, _GCH)])
        prev = (cp, bufs[i % 2], base + i * _GCH)
    pcp, pbuf, poff = prev
    pcp.wait()
    pltpu.sync_copy(pbuf, out.at[pl.ds(poff, _GCH)])


@functools.lru_cache(maxsize=None)
def _make_sc_gather():
    return functools.partial(
        pl.kernel,
        out_type=jax.ShapeDtypeStruct((B * K, H), jnp.float32),
        mesh=plsc.VectorSubcoreMesh(core_axis_name="c", subcore_axis_name="s",
                                    num_cores=_NC, num_subcores=_NS),
        scratch_types=[
            pltpu.VMEM((_RPW,), jnp.int32),
            pltpu.VMEM((_GCH, H), jnp.float32),
            pltpu.VMEM((_GCH, H), jnp.float32),
            pltpu.SemaphoreType.DMA,
            pltpu.SemaphoreType.DMA,
        ],
    )(_sc_gather_body)


def kernel(particles, log_weights):
    anc3, nlw3 = _tc_call(log_weights, log_weights.reshape(B, K, 1))
    table = particles.reshape(B * K, H)
    out_flat = _make_sc_gather()(table, anc3.reshape(B * K))
    return out_flat.reshape(B, K, H), nlw3.reshape(B, K)
